# Initial kernel scaffold; baseline (speedup 1.0000x reference)
import jax
import jax.numpy as jnp
from jax.experimental import pallas as pl


def kernel(z_e, codebook):
    raise NotImplementedError("write your pallas kernel here")


# speedup vs baseline: 3.1211x; 3.1211x over previous
import functools

import jax
import jax.numpy as jnp
import numpy as np
from jax import lax
from jax.experimental import pallas as pl
from jax.experimental.pallas import tpu as pltpu
from jax.experimental.pallas import tpu_sc as plsc

B, T, K, D = 8, 576, 512, 64
N = B * T
NIDX = 2 * N

_PERM = np.array([8 * (i % 8) + i // 8 for i in range(D)], dtype=np.int32)


def _topk_body(z_ref, c_ref, dist_ref, idx_ref):
    z = z_ref[...]
    c = c_ref[...]
    ones_col = jnp.ones((N, 1), jnp.float32)
    zaug = jnp.concatenate([z, ones_col], axis=1)
    c2 = jnp.sum(c * c, axis=1, keepdims=True)
    caug = jnp.concatenate([-2.0 * c, c2], axis=1)
    cross = lax.dot_general(zaug, caug, (((1,), (1,)), ((), ())),
                            preferred_element_type=jnp.float32,
                            precision=lax.Precision.HIGHEST)
    z2 = jnp.sum(z * z, axis=1, keepdims=True)
    dist = z2 + cross
    dist_ref[...] = dist

    iota_k = lax.broadcasted_iota(jnp.int32, (N, K), 1)
    m1 = jnp.min(dist, axis=1, keepdims=True)
    a1 = jnp.min(jnp.where(dist == m1, iota_k, K), axis=1)
    dm = jnp.where(iota_k == a1[:, None], jnp.inf, dist)
    m2 = jnp.min(dm, axis=1, keepdims=True)
    a2 = jnp.min(jnp.where(dm == m2, iota_k, K), axis=1)
    idx_ref[...] = jnp.stack([a1, a2], axis=0)


_topk_call = pl.pallas_call(
    _topk_body,
    out_shape=(
        jax.ShapeDtypeStruct((N, K), jnp.float32),
        jax.ShapeDtypeStruct((2, N), jnp.int32),
    ),
)


@functools.cache
def _gather_call():
    info = plsc.get_sparse_core_info()
    nw = info.num_cores * info.num_subcores
    rows_per_w = NIDX // nw
    mesh = plsc.VectorSubcoreMesh(core_axis_name="c", subcore_axis_name="s")

    @functools.partial(
        pl.kernel,
        mesh=mesh,
        out_type=jax.ShapeDtypeStruct((NIDX, 2 * D), jnp.float32),
        scratch_types=[
            pltpu.VMEM((rows_per_w,), jnp.int32),
            pltpu.VMEM((rows_per_w, 2 * D), jnp.float32),
            pltpu.SemaphoreType.DMA,
        ],
    )
    def gather_k(table_hbm, idx_hbm, out_hbm, idx_v, rows_v, sem):
        wid = lax.axis_index("s") * info.num_cores + lax.axis_index("c")
        base = wid * rows_per_w
        pltpu.sync_copy(idx_hbm.at[pl.ds(base, rows_per_w)], idx_v)
        pltpu.async_copy(table_hbm.at[idx_v], rows_v, sem).wait()
        pltpu.sync_copy(rows_v, out_hbm.at[pl.ds(base, rows_per_w)])

    return gather_k


def _ref_order_dist(zp, rp):
    diff = zp - rp
    prod = diff * diff
    a = prod[:, 0:32] + prod[:, 32:64]
    b = a[:, 0:16] + a[:, 16:32]
    g = b[:, 0:8] + b[:, 8:16]
    s = g[:, 0:1]
    for j in range(1, 8):
        s = s + g[:, j:j + 1]
    return s


def _refine_body(z_ref, zp_ref, rows_ref, idx_ref,
                 q_ref, ek_ref, zq_ref, perp_ref):
    zp = zp_ref[...]
    r1p = rows_ref[0:N, 0:D]
    r2p = rows_ref[N:NIDX, 0:D]
    d1 = _ref_order_dist(zp, r1p)[:, 0]
    d2 = _ref_order_dist(zp, r2p)[:, 0]
    a1 = idx_ref[0, :]
    a2 = idx_ref[1, :]
    pick2 = (d2 < d1) | ((d2 == d1) & (a2 < a1))
    q = jnp.where(pick2, a2, a1)
    q_ref[...] = q[None, :]

    z = z_ref[...]
    r1 = rows_ref[0:N, D:2 * D]
    r2 = rows_ref[N:NIDX, D:2 * D]
    ek = jnp.where(pick2[:, None], r2, r1)
    ek_ref[...] = ek
    zq_ref[...] = z + (ek - z)

    iota_k = lax.broadcasted_iota(jnp.int32, (N, K), 1)
    onehot = (q[:, None] == iota_k).astype(jnp.float32)
    counts = jnp.sum(onehot, axis=0, keepdims=True)
    avg = counts / np.float32(N)
    ent = jnp.sum(avg * jnp.log(avg + 1e-10), axis=1, keepdims=True)
    perp_ref[...] = jnp.exp(-ent)


_refine_call = pl.pallas_call(
    _refine_body,
    out_shape=(
        jax.ShapeDtypeStruct((1, N), jnp.int32),
        jax.ShapeDtypeStruct((N, D), jnp.float32),
        jax.ShapeDtypeStruct((N, D), jnp.float32),
        jax.ShapeDtypeStruct((1, 1), jnp.float32),
    ),
)


def kernel(z_e, codebook):
    z2d = z_e.reshape(N, D)
    zp = z2d[:, _PERM]
    cp = codebook[:, _PERM]
    ctab = jnp.concatenate([cp, codebook], axis=1)

    dist, idx2 = _topk_call(z2d, codebook)
    rows = _gather_call()(ctab, idx2.reshape(NIDX))
    q, ek, zq, perp = _refine_call(z2d, zp, rows, idx2)

    return (z_e, codebook,
            dist.reshape(B, T, K),
            q.reshape(B, T),
            perp.reshape(()),
            ek.reshape(B, T, D),
            zq.reshape(B, T, D))


# speedup vs baseline: 4.3258x; 1.3860x over previous
import functools

import jax
import jax.numpy as jnp
import numpy as np
from jax import lax
from jax.experimental import pallas as pl
from jax.experimental.pallas import tpu as pltpu
from jax.experimental.pallas import tpu_sc as plsc

B, T, K, D = 8, 576, 512, 64
N = B * T
NIDX = 2 * N

_PERM = np.array([8 * (i % 8) + i // 8 for i in range(D)], dtype=np.int32)


def _topk_body(z_ref, c_ref, dist_ref, idx_ref):
    z = z_ref[...]
    c = c_ref[...]
    ones_col = jnp.ones((N, 1), jnp.float32)
    zaug = jnp.concatenate([z, ones_col], axis=1)
    c2 = jnp.sum(c * c, axis=1, keepdims=True)
    caug = jnp.concatenate([-2.0 * c, c2], axis=1)
    zh = zaug.astype(jnp.bfloat16)
    zl = (zaug - zh.astype(jnp.float32)).astype(jnp.bfloat16)
    ch = caug.astype(jnp.bfloat16)
    cl = (caug - ch.astype(jnp.float32)).astype(jnp.bfloat16)
    dims = (((1,), (1,)), ((), ()))
    cross = (lax.dot_general(zh, ch, dims, preferred_element_type=jnp.float32)
             + (lax.dot_general(zh, cl, dims, preferred_element_type=jnp.float32)
                + lax.dot_general(zl, ch, dims, preferred_element_type=jnp.float32)))
    z2 = jnp.sum(z * z, axis=1, keepdims=True)
    dist = z2 + cross
    dist_ref[...] = dist

    iota_f = lax.broadcasted_iota(jnp.int32, (N, K), 1).astype(jnp.float32)
    m1 = jnp.min(dist, axis=1, keepdims=True)
    a1f = jnp.min(jnp.where(dist == m1, iota_f, np.float32(K)),
                  axis=1, keepdims=True)
    dm = jnp.where(iota_f == a1f, jnp.inf, dist)
    m2 = jnp.min(dm, axis=1, keepdims=True)
    a2f = jnp.min(jnp.where(dm == m2, iota_f, np.float32(K)),
                  axis=1, keepdims=True)
    both = jnp.concatenate([a1f, a2f], axis=1)
    idx_ref[...] = jnp.transpose(both).astype(jnp.int32)


_topk_call = pl.pallas_call(
    _topk_body,
    out_shape=(
        jax.ShapeDtypeStruct((N, K), jnp.float32),
        jax.ShapeDtypeStruct((2, N), jnp.int32),
    ),
)


@functools.cache
def _gather_call():
    info = plsc.get_sparse_core_info()
    nw = info.num_cores * info.num_subcores
    rows_per_w = NIDX // nw
    mesh = plsc.VectorSubcoreMesh(core_axis_name="c", subcore_axis_name="s")

    @functools.partial(
        pl.kernel,
        mesh=mesh,
        out_type=jax.ShapeDtypeStruct((NIDX, 2 * D), jnp.float32),
        scratch_types=[
            pltpu.VMEM((rows_per_w,), jnp.int32),
            pltpu.VMEM((rows_per_w, 2 * D), jnp.float32),
            pltpu.SemaphoreType.DMA,
        ],
    )
    def gather_k(table_hbm, idx_hbm, out_hbm, idx_v, rows_v, sem):
        wid = lax.axis_index("s") * info.num_cores + lax.axis_index("c")
        base = wid * rows_per_w
        pltpu.sync_copy(idx_hbm.at[pl.ds(base, rows_per_w)], idx_v)
        pltpu.async_copy(table_hbm.at[idx_v], rows_v, sem).wait()
        pltpu.sync_copy(rows_v, out_hbm.at[pl.ds(base, rows_per_w)])

    return gather_k


def _ref_order_dist(zpt, rpt):
    diff = zpt - rpt
    prod = diff * diff
    a = prod[0:32, :] + prod[32:64, :]
    b = a[0:16, :] + a[16:32, :]
    g = b[0:8, :] + b[8:16, :]
    s = g[0:1, :]
    for j in range(1, 8):
        s = s + g[j:j + 1, :]
    return s


def _refine_body(z_ref, zpt_ref, rows_ref, idx_ref,
                 q_ref, ek_ref, zq_ref, perp_ref):
    zpt = zpt_ref[...]
    r1pt = jnp.transpose(rows_ref[0:N, 0:D])
    r2pt = jnp.transpose(rows_ref[N:NIDX, 0:D])
    d1 = _ref_order_dist(zpt, r1pt)
    d2 = _ref_order_dist(zpt, r2pt)
    a1 = idx_ref[0:1, :]
    a2 = idx_ref[1:2, :]
    pick2 = (d2 < d1) | ((d2 == d1) & (a2 < a1))
    q = jnp.where(pick2, a2, a1)
    q_ref[...] = q

    z = z_ref[...]
    r1 = rows_ref[0:N, D:2 * D]
    r2 = rows_ref[N:NIDX, D:2 * D]
    pick_col = jnp.transpose(pick2)
    ek = jnp.where(pick_col, r2, r1)
    ek_ref[...] = ek
    zq_ref[...] = z + (ek - z)

    iota_kr = lax.broadcasted_iota(jnp.int32, (K, N), 0)
    onehot = (iota_kr == q).astype(jnp.float32)
    counts = jnp.sum(onehot, axis=1, keepdims=True)
    avg = counts / np.float32(N)
    ent = jnp.sum(avg * jnp.log(avg + 1e-10), axis=0, keepdims=True)
    perp_ref[...] = jnp.exp(-ent)


_refine_call = pl.pallas_call(
    _refine_body,
    out_shape=(
        jax.ShapeDtypeStruct((1, N), jnp.int32),
        jax.ShapeDtypeStruct((N, D), jnp.float32),
        jax.ShapeDtypeStruct((N, D), jnp.float32),
        jax.ShapeDtypeStruct((1, 1), jnp.float32),
    ),
)


def kernel(z_e, codebook):
    z2d = z_e.reshape(N, D)
    zpt = jnp.transpose(z2d[:, _PERM])
    cp = codebook[:, _PERM]
    ctab = jnp.concatenate([cp, codebook], axis=1)

    dist, idx2 = _topk_call(z2d, codebook)
    rows = _gather_call()(ctab, idx2.reshape(NIDX))
    q, ek, zq, perp = _refine_call(z2d, zpt, rows, idx2)

    return (z_e, codebook,
            dist.reshape(B, T, K),
            q.reshape(B, T),
            perp.reshape(()),
            ek.reshape(B, T, D),
            zq.reshape(B, T, D))


# speedup vs baseline: 4.4993x; 1.0401x over previous
import functools

import jax
import jax.numpy as jnp
import numpy as np
from jax import lax
from jax.experimental import pallas as pl
from jax.experimental.pallas import tpu as pltpu
from jax.experimental.pallas import tpu_sc as plsc

B, T, K, D = 8, 576, 512, 64
N = B * T
NIDX = 2 * N


def _topk_body(z_ref, c_ref, dist_ref, idx_ref, pair_ref):
    z = z_ref[...]
    c = c_ref[...]
    ones_col = jnp.ones((N, 1), jnp.float32)
    zaug = jnp.concatenate([z, ones_col], axis=1)
    c2 = jnp.sum(c * c, axis=1, keepdims=True)
    caug = jnp.concatenate([-2.0 * c, c2], axis=1)
    zh = zaug.astype(jnp.bfloat16)
    zl = (zaug - zh.astype(jnp.float32)).astype(jnp.bfloat16)
    ch = caug.astype(jnp.bfloat16)
    cl = (caug - ch.astype(jnp.float32)).astype(jnp.bfloat16)
    dims = (((1,), (1,)), ((), ()))
    cross = (lax.dot_general(zh, ch, dims, preferred_element_type=jnp.float32)
             + (lax.dot_general(zh, cl, dims, preferred_element_type=jnp.float32)
                + lax.dot_general(zl, ch, dims, preferred_element_type=jnp.float32)))
    z2 = jnp.sum(z * z, axis=1, keepdims=True)
    dist = z2 + cross
    dist_ref[...] = dist

    iota_f = lax.broadcasted_iota(jnp.int32, (N, K), 1).astype(jnp.float32)
    m1 = jnp.min(dist, axis=1, keepdims=True)
    a1f = jnp.min(jnp.where(dist == m1, iota_f, np.float32(K)),
                  axis=1, keepdims=True)
    dm = jnp.where(iota_f == a1f, jnp.inf, dist)
    m2 = jnp.min(dm, axis=1, keepdims=True)
    a2f = jnp.min(jnp.where(dm == m2, iota_f, np.float32(K)),
                  axis=1, keepdims=True)
    both = jnp.concatenate([a1f, a2f], axis=1)
    idx = jnp.transpose(both).astype(jnp.int32)
    idx_ref[...] = idx
    pair_ref[...] = jnp.concatenate([c, c], axis=1)


_topk_call = pl.pallas_call(
    _topk_body,
    out_shape=(
        jax.ShapeDtypeStruct((N, K), jnp.float32),
        jax.ShapeDtypeStruct((2, N), jnp.int32),
        jax.ShapeDtypeStruct((K, 2 * D), jnp.float32),
    ),
)


@functools.cache
def _gather_call():
    info = plsc.get_sparse_core_info()
    nw = info.num_cores * info.num_subcores
    rows_per_w = NIDX // nw
    mesh = plsc.VectorSubcoreMesh(core_axis_name="c", subcore_axis_name="s")

    @functools.partial(
        pl.kernel,
        mesh=mesh,
        out_type=jax.ShapeDtypeStruct((NIDX, 2 * D), jnp.float32),
        scratch_types=[
            pltpu.VMEM((rows_per_w,), jnp.int32),
            pltpu.VMEM((rows_per_w, 2 * D), jnp.float32),
            pltpu.SemaphoreType.DMA,
        ],
    )
    def gather_k(table_hbm, idx_hbm, out_hbm, idx_v, rows_v, sem):
        wid = lax.axis_index("s") * info.num_cores + lax.axis_index("c")
        base = wid * rows_per_w
        pltpu.sync_copy(idx_hbm.at[pl.ds(base, rows_per_w)], idx_v)
        pltpu.async_copy(table_hbm.at[idx_v], rows_v, sem).wait()
        pltpu.sync_copy(rows_v, out_hbm.at[pl.ds(base, rows_per_w)])

    return gather_k


def _ref_order_dist(zt, rt):
    diff = zt - rt
    prod = (diff * diff).reshape(8, 8, N)
    a = prod[:, 0:4, :] + prod[:, 4:8, :]
    b = a[:, 0:2, :] + a[:, 2:4, :]
    g = (b[:, 0:1, :] + b[:, 1:2, :]).reshape(8, N)
    s = g[0:1, :]
    for j in range(1, 8):
        s = s + g[j:j + 1, :]
    return s


def _refine_body(z_ref, rows_ref, idx_ref,
                 q_ref, ek_ref, zq_ref, perp_ref):
    z = z_ref[...]
    zt = jnp.transpose(z)
    a1 = idx_ref[0:1, :]
    a2 = idx_ref[1:2, :]
    r1 = rows_ref[0:N, 0:D]
    r2 = rows_ref[N:NIDX, 0:D]
    d1 = _ref_order_dist(zt, jnp.transpose(r1))
    d2 = _ref_order_dist(zt, jnp.transpose(r2))
    pick2 = (d2 < d1) | ((d2 == d1) & (a2 < a1))
    q = jnp.where(pick2, a2, a1)
    q_ref[...] = q

    pick_col = jnp.transpose(pick2)
    ek = jnp.where(pick_col, r2, r1)
    ek_ref[...] = ek
    zq_ref[...] = z + (ek - z)

    iota_kr = lax.broadcasted_iota(jnp.int32, (K, N), 0)
    onehot = (iota_kr == q).astype(jnp.float32)
    counts = jnp.sum(onehot, axis=1, keepdims=True)
    avg = counts / np.float32(N)
    ent = jnp.sum(avg * jnp.log(avg + 1e-10), axis=0, keepdims=True)
    perp_ref[...] = jnp.exp(-ent)


_refine_call = pl.pallas_call(
    _refine_body,
    out_shape=(
        jax.ShapeDtypeStruct((1, N), jnp.int32),
        jax.ShapeDtypeStruct((N, D), jnp.float32),
        jax.ShapeDtypeStruct((N, D), jnp.float32),
        jax.ShapeDtypeStruct((1, 1), jnp.float32),
    ),
)


def kernel(z_e, codebook):
    z2d = z_e.reshape(N, D)
    dist, idx2, pairtab = _topk_call(z2d, codebook)
    rows = _gather_call()(pairtab, idx2.reshape(NIDX))
    q, ek, zq, perp = _refine_call(z2d, rows, idx2)

    return (z_e, codebook,
            dist.reshape(B, T, K),
            q.reshape(B, T),
            perp.reshape(()),
            ek.reshape(B, T, D),
            zq.reshape(B, T, D))
